# baseline (speedup 1.0000x reference)
import functools
import math

import jax
import jax.numpy as jnp
from jax import lax
from jax.experimental import pallas as pl
from jax.experimental.pallas import tpu as pltpu
from jax.experimental.pallas import tpu_sc as plsc

_N = 10000
_D = 128
_ALPHA = 0.1
_LAMBDA = 1.0

_NC = 2
_NS = 16
_NW = _NC * _NS
_C = 128

_NPAD = 10112
_C0 = 40
_C1 = 120
_ZROWS = _NPAD // _NS
def _ceil_to(x, m):
    return (x + m - 1) // m * m


def _sc_degrees(dst3):
    ch = dst3.shape[1]
    mesh = plsc.VectorSubcoreMesh(core_axis_name="c", subcore_axis_name="s")

    @functools.partial(
        pl.kernel,
        mesh=mesh,
        out_type=jax.ShapeDtypeStruct((_NC, _NPAD, _D), jnp.float32),
        scratch_types=[
            pltpu.VMEM((ch, _C), jnp.int32),
            pltpu.VMEM((_C, _D), jnp.float32),
            pltpu.VMEM((8, _D), jnp.float32),
            pltpu.VMEM_SHARED((_NPAD, _D), jnp.float32),
        ],
    )
    def k(dst_hbm, out_hbm, idx_v, ones_v, z_v, deg_sh):
        cid = lax.axis_index("c")
        sid = lax.axis_index("s")
        wid = cid * _NS + sid

        pltpu.sync_copy(dst_hbm.at[wid], idx_v)

        ones16 = jnp.ones((16,), jnp.float32)
        zeros16 = jnp.zeros((16,), jnp.float32)

        def fill_ones(i, _):
            r = i // 8
            p = i - r * 8
            ones_v[r, pl.ds(p * 16, 16)] = ones16
            return 0

        lax.fori_loop(0, _C * 8, fill_ones, 0)

        def fill_zero(i, _):
            for p in range(_D // 16):
                z_v[i, pl.ds(p * 16, 16)] = zeros16
            return 0

        lax.fori_loop(0, 8, fill_zero, 0)

        base = sid * _ZROWS

        def zero_sh(q, _):
            pltpu.sync_copy(z_v, deg_sh.at[pl.ds(base + q * 8, 8)])
            return 0

        lax.fori_loop(0, _ZROWS // 8, zero_sh, 0)
        plsc.subcore_barrier()

        def body(j, _):
            pltpu.sync_copy(ones_v, deg_sh.at[idx_v.at[j]], add=True)
            return 0

        lax.fori_loop(0, ch, body, 0)
        plsc.subcore_barrier()

        pltpu.sync_copy(deg_sh.at[pl.ds(base, _ZROWS)],
                        out_hbm.at[cid, pl.ds(base, _ZROWS)])

    return k(dst3)


def _sc_aggregate(hn, src3, dst3, c0, c1):
    w = 40
    mesh = plsc.VectorSubcoreMesh(core_axis_name="c", subcore_axis_name="s")

    @functools.partial(
        pl.kernel,
        mesh=mesh,
        out_type=jax.ShapeDtypeStruct((_NC, _NPAD, _D), jnp.float32),
        scratch_types=[
            pltpu.VMEM((w, _C), jnp.int32),
            pltpu.VMEM((w, _C), jnp.int32),
            pltpu.VMEM((_C, _D), jnp.float32),
            pltpu.VMEM((_C, _D), jnp.float32),
            pltpu.VMEM((8, _D), jnp.float32),
            pltpu.VMEM_SHARED((_NPAD, _D), jnp.float32),
            pltpu.SemaphoreType.DMA,
            pltpu.SemaphoreType.DMA,
            pltpu.SemaphoreType.DMA,
            pltpu.SemaphoreType.DMA,
        ],
    )
    def k(hn_hbm, src_hbm, dst_hbm, out_hbm,
          sidx_v, didx_v, rows0, rows1, z_v, agg_sh, gs0, gs1, ss0, ss1):
        cid = lax.axis_index("c")
        sid = lax.axis_index("s")
        wid = cid * _NS + sid
        nwin = lax.select(cid == 0, c0 // w, c1 // w)

        def fill_zero(i, _):
            for p in range(_D // 16):
                z_v[i, pl.ds(p * 16, 16)] = jnp.zeros((16,), jnp.float32)
            return 0

        lax.fori_loop(0, 8, fill_zero, 0)

        base = sid * _ZROWS

        def zero_sh(q, _):
            pltpu.sync_copy(z_v, agg_sh.at[pl.ds(base + q * 8, 8)])
            return 0

        lax.fori_loop(0, _ZROWS // 8, zero_sh, 0)
        plsc.subcore_barrier()

        def window(h, _):
            pltpu.sync_copy(src_hbm.at[wid, pl.ds(h * w, w)], sidx_v)
            pltpu.sync_copy(dst_hbm.at[wid, pl.ds(h * w, w)], didx_v)
            pltpu.async_copy(hn_hbm.at[sidx_v.at[0]], rows0, gs0)
            pltpu.async_copy(hn_hbm.at[sidx_v.at[1]], rows1, gs1)

            def body(i, _):
                j0 = 2 * i
                j1 = 2 * i + 1
                n0 = lax.rem(j0 + 2, w)
                n1 = lax.rem(j1 + 2, w)
                pltpu.make_async_copy(hn_hbm.at[sidx_v.at[j0]], rows0, gs0).wait()
                pltpu.async_copy(rows0, agg_sh.at[didx_v.at[j0]], ss0, add=True)
                pltpu.make_async_copy(hn_hbm.at[sidx_v.at[j1]], rows1, gs1).wait()
                pltpu.async_copy(rows1, agg_sh.at[didx_v.at[j1]], ss1, add=True)
                pltpu.make_async_copy(rows0, agg_sh.at[didx_v.at[j0]], ss0).wait()
                pltpu.async_copy(hn_hbm.at[sidx_v.at[n0]], rows0, gs0)
                pltpu.make_async_copy(rows1, agg_sh.at[didx_v.at[j1]], ss1).wait()
                pltpu.async_copy(hn_hbm.at[sidx_v.at[n1]], rows1, gs1)
                return 0

            lax.fori_loop(0, w // 2, body, 0)
            pltpu.make_async_copy(hn_hbm.at[sidx_v.at[0]], rows0, gs0).wait()
            pltpu.make_async_copy(hn_hbm.at[sidx_v.at[1]], rows1, gs1).wait()
            return 0

        lax.fori_loop(0, nwin, window, 0)
        plsc.subcore_barrier()

        pltpu.sync_copy(agg_sh.at[pl.ds(base, _ZROWS)],
                        out_hbm.at[cid, pl.ds(base, _ZROWS)])

    return k(hn, src3, dst3)


_R = 2000


def _fc_body(x_ref, w_ref, b_ref, o_ref):
    x = x_ref[...]
    w = w_ref[...]
    o = lax.dot_general(x, w, (((1,), (1,)), ((), ())),
                        preferred_element_type=jnp.float32)
    o_ref[...] = o + b_ref[...]


def _tc_fc(feat, w_fc, b_fc):
    grid = (_N // _R,)
    return pl.pallas_call(
        _fc_body,
        grid=grid,
        in_specs=[
            pl.BlockSpec((_R, _D), lambda i: (i, 0)),
            pl.BlockSpec((_D, _D), lambda i: (0, 0)),
            pl.BlockSpec((1, _D), lambda i: (0, 0)),
        ],
        out_specs=pl.BlockSpec((_R, _D), lambda i: (i, 0)),
        out_shape=jax.ShapeDtypeStruct((_N, _D), jnp.float32),
    )(feat, w_fc, b_fc.reshape(1, _D))


def _prep_body(h0_ref, d_ref, norm_ref, hn_ref):
    deg = d_ref[0, :, 0:1] + d_ref[1, :, 0:1]
    norm = lax.rsqrt(jnp.maximum(deg, 1.0))
    norm_ref[...] = norm
    hn_ref[...] = h0_ref[...] * norm


def _tc_prep(h0, deg2):
    grid = (_N // _R,)
    return pl.pallas_call(
        _prep_body,
        grid=grid,
        in_specs=[
            pl.BlockSpec((_R, _D), lambda i: (i, 0)),
            pl.BlockSpec((2, _R, _D), lambda i: (0, i, 0)),
        ],
        out_specs=[
            pl.BlockSpec((_R, 1), lambda i: (i, 0)),
            pl.BlockSpec((_R, _D), lambda i: (i, 0)),
        ],
        out_shape=[
            jax.ShapeDtypeStruct((_N, 1), jnp.float32),
            jax.ShapeDtypeStruct((_N, _D), jnp.float32),
        ],
    )(h0, deg2)


def _layer_core(agg_ref, norm_ref, f0_ref, w_ref, b_ref, beta):
    a = agg_ref[0] + agg_ref[1]
    norm = norm_ref[...]
    h = a * norm * (1.0 - _ALPHA) + _ALPHA * f0_ref[...]
    row = lax.broadcasted_iota(jnp.int32, (_D, _D), 0)
    col = lax.broadcasted_iota(jnp.int32, (_D, _D), 1)
    eye = jnp.where(row == col, 1.0 - beta, 0.0).astype(jnp.float32)
    wmod = eye + beta * w_ref[...]
    o = lax.dot_general(h, wmod, (((1,), (0,)), ((), ())),
                        preferred_element_type=jnp.float32)
    return jnp.maximum(o + b_ref[...], 0.0), norm


def _layer1_body(agg_ref, norm_ref, f0_ref, w_ref, b_ref, o_ref, hn_ref, *,
                 beta):
    o, norm = _layer_core(agg_ref, norm_ref, f0_ref, w_ref, b_ref, beta)
    o_ref[...] = o
    hn_ref[...] = o * norm


def _layer2_body(agg_ref, norm_ref, f0_ref, w_ref, b_ref, o_ref, *, beta):
    o, _ = _layer_core(agg_ref, norm_ref, f0_ref, w_ref, b_ref, beta)
    o_ref[...] = o


def _tc_layer(agg2, norm, f0, w, b, beta, want_hn):
    grid = (_N // _R,)
    in_specs = [
        pl.BlockSpec((2, _R, _D), lambda i: (0, i, 0)),
        pl.BlockSpec((_R, 1), lambda i: (i, 0)),
        pl.BlockSpec((_R, _D), lambda i: (i, 0)),
        pl.BlockSpec((_D, _D), lambda i: (0, 0)),
        pl.BlockSpec((1, _D), lambda i: (0, 0)),
    ]
    if want_hn:
        return pl.pallas_call(
            functools.partial(_layer1_body, beta=beta),
            grid=grid,
            in_specs=in_specs,
            out_specs=[
                pl.BlockSpec((_R, _D), lambda i: (i, 0)),
                pl.BlockSpec((_R, _D), lambda i: (i, 0)),
            ],
            out_shape=[
                jax.ShapeDtypeStruct((_N, _D), jnp.float32),
                jax.ShapeDtypeStruct((_N, _D), jnp.float32),
            ],
        )(agg2, norm, f0, w, b.reshape(1, _D))
    return pl.pallas_call(
        functools.partial(_layer2_body, beta=beta),
        grid=grid,
        in_specs=in_specs,
        out_specs=pl.BlockSpec((_R, _D), lambda i: (i, 0)),
        out_shape=jax.ShapeDtypeStruct((_N, _D), jnp.float32),
    )(agg2, norm, f0, w, b.reshape(1, _D))


@jax.jit
def kernel(feat, edge_index, W_fc, b_fc, W1, b1, W2, b2):
    e = edge_index.shape[1]
    ch = _ceil_to(_ceil_to(e, _NW * _C) // (_NW * _C), 2)
    e_pad = _NW * ch * _C

    src = edge_index[0]
    dst = edge_index[1]
    pad = e_pad - e
    src_p = jnp.concatenate([src, jnp.zeros((pad,), jnp.int32)])
    dst_p = jnp.concatenate([dst, jnp.full((pad,), _N, jnp.int32)])
    src3 = src_p.reshape(_NW, ch, _C)
    dst3 = dst_p.reshape(_NW, ch, _C)

    c0, c1 = _C0, _C1
    cmax = max(c0, c1)
    n0 = _NS * c0 * _C

    def slab(flat, fill):
        a = jnp.full((_NW, cmax, _C), fill, jnp.int32)
        a = a.at[:_NS, :c0].set(flat[:n0].reshape(_NS, c0, _C))
        a = a.at[_NS:, :c1].set(flat[n0:].reshape(_NS, c1, _C))
        return a

    src3a = slab(src_p, 0)
    dst3a = slab(dst_p, _N)

    deg2 = _sc_degrees(dst3)
    h0 = _tc_fc(feat, W_fc, b_fc)
    norm, hn1 = _tc_prep(h0, deg2)

    beta1 = math.log(_LAMBDA / 1.0 + 1.0)
    beta2 = math.log(_LAMBDA / 2.0 + 1.0)

    agg1 = _sc_aggregate(hn1, src3a, dst3a, c0, c1)
    res, hn2 = _tc_layer(agg1, norm, h0, W1, b1, beta1, True)
    agg2 = _sc_aggregate(hn2, src3a, dst3a, c0, c1)
    out = _tc_layer(agg2, norm, h0, W2, b2, beta2, False)
    return out


# speedup vs baseline: 1.0673x; 1.0673x over previous
import functools
import math

import jax
import jax.numpy as jnp
from jax import lax
from jax.experimental import pallas as pl
from jax.experimental.pallas import tpu as pltpu
from jax.experimental.pallas import tpu_sc as plsc

_N = 10000
_D = 128
_ALPHA = 0.1
_LAMBDA = 1.0

_NC = 2
_NS = 16
_NW = _NC * _NS
_C = 128

_NPAD = 10112
_C0 = 120
_C1 = 40
_ZROWS = _NPAD // _NS
def _ceil_to(x, m):
    return (x + m - 1) // m * m


def _sc_degrees(dst3):
    ch = dst3.shape[1]
    mesh = plsc.VectorSubcoreMesh(core_axis_name="c", subcore_axis_name="s")

    @functools.partial(
        pl.kernel,
        mesh=mesh,
        out_type=jax.ShapeDtypeStruct((_NC, _NPAD, _D), jnp.float32),
        scratch_types=[
            pltpu.VMEM((ch, _C), jnp.int32),
            pltpu.VMEM((_C, _D), jnp.float32),
            pltpu.VMEM((8, _D), jnp.float32),
            pltpu.VMEM_SHARED((_NPAD, _D), jnp.float32),
        ],
    )
    def k(dst_hbm, out_hbm, idx_v, ones_v, z_v, deg_sh):
        cid = lax.axis_index("c")
        sid = lax.axis_index("s")
        wid = cid * _NS + sid

        pltpu.sync_copy(dst_hbm.at[wid], idx_v)

        ones16 = jnp.ones((16,), jnp.float32)
        zeros16 = jnp.zeros((16,), jnp.float32)

        def fill_ones(i, _):
            r = i // 8
            p = i - r * 8
            ones_v[r, pl.ds(p * 16, 16)] = ones16
            return 0

        lax.fori_loop(0, _C * 8, fill_ones, 0)

        def fill_zero(i, _):
            for p in range(_D // 16):
                z_v[i, pl.ds(p * 16, 16)] = zeros16
            return 0

        lax.fori_loop(0, 8, fill_zero, 0)

        base = sid * _ZROWS

        def zero_sh(q, _):
            pltpu.sync_copy(z_v, deg_sh.at[pl.ds(base + q * 8, 8)])
            return 0

        lax.fori_loop(0, _ZROWS // 8, zero_sh, 0)
        plsc.subcore_barrier()

        def body(j, _):
            pltpu.sync_copy(ones_v, deg_sh.at[idx_v.at[j]], add=True)
            return 0

        lax.fori_loop(0, ch, body, 0)
        plsc.subcore_barrier()

        pltpu.sync_copy(deg_sh.at[pl.ds(base, _ZROWS)],
                        out_hbm.at[cid, pl.ds(base, _ZROWS)])

    return k(dst3)


def _sc_aggregate(hn, src3, dst3, c0, c1):
    w = 40
    mesh = plsc.VectorSubcoreMesh(core_axis_name="c", subcore_axis_name="s")

    @functools.partial(
        pl.kernel,
        mesh=mesh,
        out_type=jax.ShapeDtypeStruct((_NC, _NPAD, _D), jnp.float32),
        scratch_types=[
            pltpu.VMEM((w, _C), jnp.int32),
            pltpu.VMEM((w, _C), jnp.int32),
            pltpu.VMEM((_C, _D), jnp.float32),
            pltpu.VMEM((_C, _D), jnp.float32),
            pltpu.VMEM((8, _D), jnp.float32),
            pltpu.VMEM_SHARED((_NPAD, _D), jnp.float32),
            pltpu.SemaphoreType.DMA,
            pltpu.SemaphoreType.DMA,
            pltpu.SemaphoreType.DMA,
            pltpu.SemaphoreType.DMA,
        ],
    )
    def k(hn_hbm, src_hbm, dst_hbm, out_hbm,
          sidx_v, didx_v, rows0, rows1, z_v, agg_sh, gs0, gs1, ss0, ss1):
        cid = lax.axis_index("c")
        sid = lax.axis_index("s")
        wid = cid * _NS + sid
        nwin = lax.select(cid == 0, c0 // w, c1 // w)

        def fill_zero(i, _):
            for p in range(_D // 16):
                z_v[i, pl.ds(p * 16, 16)] = jnp.zeros((16,), jnp.float32)
            return 0

        lax.fori_loop(0, 8, fill_zero, 0)

        base = sid * _ZROWS

        def zero_sh(q, _):
            pltpu.sync_copy(z_v, agg_sh.at[pl.ds(base + q * 8, 8)])
            return 0

        lax.fori_loop(0, _ZROWS // 8, zero_sh, 0)
        plsc.subcore_barrier()

        def window(h, _):
            pltpu.sync_copy(src_hbm.at[wid, pl.ds(h * w, w)], sidx_v)
            pltpu.sync_copy(dst_hbm.at[wid, pl.ds(h * w, w)], didx_v)
            pltpu.async_copy(hn_hbm.at[sidx_v.at[0]], rows0, gs0)
            pltpu.async_copy(hn_hbm.at[sidx_v.at[1]], rows1, gs1)

            def body(i, _):
                j0 = 2 * i
                j1 = 2 * i + 1
                n0 = lax.rem(j0 + 2, w)
                n1 = lax.rem(j1 + 2, w)
                pltpu.make_async_copy(hn_hbm.at[sidx_v.at[j0]], rows0, gs0).wait()
                pltpu.async_copy(rows0, agg_sh.at[didx_v.at[j0]], ss0, add=True)
                pltpu.make_async_copy(hn_hbm.at[sidx_v.at[j1]], rows1, gs1).wait()
                pltpu.async_copy(rows1, agg_sh.at[didx_v.at[j1]], ss1, add=True)
                pltpu.make_async_copy(rows0, agg_sh.at[didx_v.at[j0]], ss0).wait()
                pltpu.async_copy(hn_hbm.at[sidx_v.at[n0]], rows0, gs0)
                pltpu.make_async_copy(rows1, agg_sh.at[didx_v.at[j1]], ss1).wait()
                pltpu.async_copy(hn_hbm.at[sidx_v.at[n1]], rows1, gs1)
                return 0

            lax.fori_loop(0, w // 2, body, 0)
            pltpu.make_async_copy(hn_hbm.at[sidx_v.at[0]], rows0, gs0).wait()
            pltpu.make_async_copy(hn_hbm.at[sidx_v.at[1]], rows1, gs1).wait()
            return 0

        lax.fori_loop(0, nwin, window, 0)
        plsc.subcore_barrier()

        pltpu.sync_copy(agg_sh.at[pl.ds(base, _ZROWS)],
                        out_hbm.at[cid, pl.ds(base, _ZROWS)])

    return k(hn, src3, dst3)


_R = 2000


def _fc_body(x_ref, w_ref, b_ref, o_ref):
    x = x_ref[...]
    w = w_ref[...]
    o = lax.dot_general(x, w, (((1,), (1,)), ((), ())),
                        preferred_element_type=jnp.float32)
    o_ref[...] = o + b_ref[...]


def _tc_fc(feat, w_fc, b_fc):
    grid = (_N // _R,)
    return pl.pallas_call(
        _fc_body,
        grid=grid,
        in_specs=[
            pl.BlockSpec((_R, _D), lambda i: (i, 0)),
            pl.BlockSpec((_D, _D), lambda i: (0, 0)),
            pl.BlockSpec((1, _D), lambda i: (0, 0)),
        ],
        out_specs=pl.BlockSpec((_R, _D), lambda i: (i, 0)),
        out_shape=jax.ShapeDtypeStruct((_N, _D), jnp.float32),
    )(feat, w_fc, b_fc.reshape(1, _D))


def _prep_body(h0_ref, d_ref, norm_ref, hn_ref):
    deg = d_ref[0, :, 0:1] + d_ref[1, :, 0:1]
    norm = lax.rsqrt(jnp.maximum(deg, 1.0))
    norm_ref[...] = norm
    hn_ref[...] = h0_ref[...] * norm


def _tc_prep(h0, deg2):
    grid = (_N // _R,)
    return pl.pallas_call(
        _prep_body,
        grid=grid,
        in_specs=[
            pl.BlockSpec((_R, _D), lambda i: (i, 0)),
            pl.BlockSpec((2, _R, _D), lambda i: (0, i, 0)),
        ],
        out_specs=[
            pl.BlockSpec((_R, 1), lambda i: (i, 0)),
            pl.BlockSpec((_R, _D), lambda i: (i, 0)),
        ],
        out_shape=[
            jax.ShapeDtypeStruct((_N, 1), jnp.float32),
            jax.ShapeDtypeStruct((_N, _D), jnp.float32),
        ],
    )(h0, deg2)


def _layer_core(agg_ref, norm_ref, f0_ref, w_ref, b_ref, beta):
    a = agg_ref[0] + agg_ref[1]
    norm = norm_ref[...]
    h = a * norm * (1.0 - _ALPHA) + _ALPHA * f0_ref[...]
    row = lax.broadcasted_iota(jnp.int32, (_D, _D), 0)
    col = lax.broadcasted_iota(jnp.int32, (_D, _D), 1)
    eye = jnp.where(row == col, 1.0 - beta, 0.0).astype(jnp.float32)
    wmod = eye + beta * w_ref[...]
    o = lax.dot_general(h, wmod, (((1,), (0,)), ((), ())),
                        preferred_element_type=jnp.float32)
    return jnp.maximum(o + b_ref[...], 0.0), norm


def _layer1_body(agg_ref, norm_ref, f0_ref, w_ref, b_ref, o_ref, hn_ref, *,
                 beta):
    o, norm = _layer_core(agg_ref, norm_ref, f0_ref, w_ref, b_ref, beta)
    o_ref[...] = o
    hn_ref[...] = o * norm


def _layer2_body(agg_ref, norm_ref, f0_ref, w_ref, b_ref, o_ref, *, beta):
    o, _ = _layer_core(agg_ref, norm_ref, f0_ref, w_ref, b_ref, beta)
    o_ref[...] = o


def _tc_layer(agg2, norm, f0, w, b, beta, want_hn):
    grid = (_N // _R,)
    in_specs = [
        pl.BlockSpec((2, _R, _D), lambda i: (0, i, 0)),
        pl.BlockSpec((_R, 1), lambda i: (i, 0)),
        pl.BlockSpec((_R, _D), lambda i: (i, 0)),
        pl.BlockSpec((_D, _D), lambda i: (0, 0)),
        pl.BlockSpec((1, _D), lambda i: (0, 0)),
    ]
    if want_hn:
        return pl.pallas_call(
            functools.partial(_layer1_body, beta=beta),
            grid=grid,
            in_specs=in_specs,
            out_specs=[
                pl.BlockSpec((_R, _D), lambda i: (i, 0)),
                pl.BlockSpec((_R, _D), lambda i: (i, 0)),
            ],
            out_shape=[
                jax.ShapeDtypeStruct((_N, _D), jnp.float32),
                jax.ShapeDtypeStruct((_N, _D), jnp.float32),
            ],
        )(agg2, norm, f0, w, b.reshape(1, _D))
    return pl.pallas_call(
        functools.partial(_layer2_body, beta=beta),
        grid=grid,
        in_specs=in_specs,
        out_specs=pl.BlockSpec((_R, _D), lambda i: (i, 0)),
        out_shape=jax.ShapeDtypeStruct((_N, _D), jnp.float32),
    )(agg2, norm, f0, w, b.reshape(1, _D))


@jax.jit
def kernel(feat, edge_index, W_fc, b_fc, W1, b1, W2, b2):
    e = edge_index.shape[1]
    ch = _ceil_to(_ceil_to(e, _NW * _C) // (_NW * _C), 2)
    e_pad = _NW * ch * _C

    src = edge_index[0]
    dst = edge_index[1]
    pad = e_pad - e
    src_p = jnp.concatenate([src, jnp.zeros((pad,), jnp.int32)])
    dst_p = jnp.concatenate([dst, jnp.full((pad,), _N, jnp.int32)])
    src3 = src_p.reshape(_NW, ch, _C)
    dst3 = dst_p.reshape(_NW, ch, _C)

    c0, c1 = _C0, _C1
    cmax = max(c0, c1)
    n0 = _NS * c0 * _C

    def slab(flat, fill):
        a = jnp.full((_NW, cmax, _C), fill, jnp.int32)
        a = a.at[:_NS, :c0].set(flat[:n0].reshape(_NS, c0, _C))
        a = a.at[_NS:, :c1].set(flat[n0:].reshape(_NS, c1, _C))
        return a

    src3a = slab(src_p, 0)
    dst3a = slab(dst_p, _N)

    deg2 = _sc_degrees(dst3)
    h0 = _tc_fc(feat, W_fc, b_fc)
    norm, hn1 = _tc_prep(h0, deg2)

    beta1 = math.log(_LAMBDA / 1.0 + 1.0)
    beta2 = math.log(_LAMBDA / 2.0 + 1.0)

    agg1 = _sc_aggregate(hn1, src3a, dst3a, c0, c1)
    res, hn2 = _tc_layer(agg1, norm, h0, W1, b1, beta1, True)
    agg2 = _sc_aggregate(hn2, src3a, dst3a, c0, c1)
    out = _tc_layer(agg2, norm, h0, W2, b2, beta2, False)
    return out
